# baseline (speedup 1.0000x reference)
import functools

import numpy as np
import jax
import jax.numpy as jnp
from jax import lax
from jax.experimental import pallas as pl
from jax.experimental.pallas import tpu as pltpu
from jax.experimental.pallas import tpu_sc as plsc

V = 1000000
D = 64
B = 4096
L = 200
MAXLEN = 512
EPS = 1e-5

N = B * L

NC, NS = 2, 16
NW = NC * NS
CH = 128
PER_W = N // NW
IDX_ROWS = PER_W // CH
IDX_BLK = 8
N_OUTER = IDX_ROWS // IDX_BLK


def _sinusoidal_pe(max_len, d):
    pos = np.arange(max_len, dtype=np.float32)[:, None]
    div = np.exp(np.arange(0, d, 2, dtype=np.float32) * (-np.log(10000.0) / d))
    pe = np.zeros((max_len, d), dtype=np.float32)
    pe[:, 0::2] = np.sin(pos * div)
    pe[:, 1::2] = np.cos(pos * div)
    return pe


R = 1600
G = N // R
_PE_TILED = jnp.asarray(np.tile(_sinusoidal_pe(MAXLEN, D)[:L], (R // L, 1)))


def _sc_gather_body(idx_hbm, table_hbm, out_hbm, idx_v, rows_v, sem):
    wid = lax.axis_index("s") * NC + lax.axis_index("c")
    row0 = wid * IDX_ROWS

    def outer(j, carry):
        pltpu.sync_copy(idx_hbm.at[pl.ds(row0 + j * IDX_BLK, IDX_BLK)], idx_v)
        for bb in range(IDX_BLK):
            pltpu.async_copy(table_hbm.at[idx_v.at[bb]], rows_v, sem).wait()
            off = (row0 + j * IDX_BLK + bb) * CH
            pltpu.sync_copy(rows_v, out_hbm.at[pl.ds(off, CH)])
        return carry

    lax.fori_loop(0, N_OUTER, outer, 0)


@functools.cache
def _sc_gather_kernel():
    return functools.partial(
        pl.kernel,
        mesh=plsc.VectorSubcoreMesh(core_axis_name="c", subcore_axis_name="s"),
        out_type=jax.ShapeDtypeStruct((N, D), jnp.float32),
        scratch_types=[
            pltpu.VMEM((IDX_BLK, CH), jnp.int32),
            pltpu.VMEM((CH, D), jnp.float32),
            pltpu.SemaphoreType.DMA,
        ],
        compiler_params=pltpu.CompilerParams(use_tc_tiling_on_sc=False),
    )(_sc_gather_body)


def _nrm(v):
    return jnp.sqrt(jnp.clip(jnp.sum(v * v, axis=-1, keepdims=True), 1e-15))


def _artanh(z):
    z = jnp.clip(z, -1.0 + 1e-7, 1.0 - 1e-7)
    return 0.5 * jnp.log((1.0 + z) / (1.0 - z))


def _projn(y):
    n = _nrm(y)
    maxn = 1.0 - 1e-5
    return jnp.where(n > maxn, y / n * maxn, y)


def _tc_body(emb_ref, pe_ref, w_ref, b_ref, g_ref, be_ref, out_ref):
    e = emb_ref[...] + pe_ref[...]
    n = _nrm(e)
    h = _projn(jnp.tanh(n) * e / n)
    xn = _nrm(h)
    mx = jnp.dot(h, w_ref[...].T, preferred_element_type=jnp.float32)
    mxn = _nrm(mx)
    h = _projn(jnp.tanh(mxn / xn * _artanh(xn)) * mx / mxn)
    bv = b_ref[...]
    bn = _nrm(bv)
    bh = _projn(jnp.tanh(bn) * bv / bn)
    xy = jnp.sum(h * bh, axis=-1, keepdims=True)
    x2 = jnp.sum(h * h, axis=-1, keepdims=True)
    y2 = jnp.sum(bh * bh, axis=-1, keepdims=True)
    num = (1.0 + 2.0 * xy + y2) * h + (1.0 - x2) * bh
    den = 1.0 + 2.0 * xy + x2 * y2
    h = _projn(num / jnp.clip(den, 1e-15))
    n = _nrm(h)
    h = _artanh(n) * h / n
    mu = jnp.mean(h, axis=-1, keepdims=True)
    var = jnp.mean((h - mu) ** 2, axis=-1, keepdims=True)
    out_ref[...] = (h - mu) / jnp.sqrt(var + EPS) * g_ref[...] + be_ref[...]


def _tc_compute(emb, pe_t, W, b2, g2, be2):
    return pl.pallas_call(
        _tc_body,
        grid=(G,),
        in_specs=[
            pl.BlockSpec((R, D), lambda i: (i, 0)),
            pl.BlockSpec((R, D), lambda i: (0, 0)),
            pl.BlockSpec((D, D), lambda i: (0, 0)),
            pl.BlockSpec((1, D), lambda i: (0, 0)),
            pl.BlockSpec((1, D), lambda i: (0, 0)),
            pl.BlockSpec((1, D), lambda i: (0, 0)),
        ],
        out_specs=pl.BlockSpec((R, D), lambda i: (i, 0)),
        out_shape=jax.ShapeDtypeStruct((N, D), jnp.float32),
        compiler_params=pltpu.CompilerParams(
            dimension_semantics=("arbitrary",),
        ),
    )(emb, pe_t, W, b2, g2, be2)


def kernel(x, table, W, b, gamma, beta):
    xf = x.reshape(N // CH, CH).astype(jnp.int32)
    emb = _sc_gather_kernel()(xf, table)
    out = _tc_compute(
        emb,
        _PE_TILED,
        W,
        b.reshape(1, D),
        gamma.reshape(1, D),
        beta.reshape(1, D),
    )
    return out.reshape(B, L, D)


# speedup vs baseline: 1.6074x; 1.6074x over previous
import functools

import numpy as np
import jax
import jax.numpy as jnp
from jax import lax
from jax.experimental import pallas as pl
from jax.experimental.pallas import tpu as pltpu
from jax.experimental.pallas import tpu_sc as plsc

V = 1000000
D = 64
B = 4096
L = 200
MAXLEN = 512
EPS = 1e-5

N = B * L

NC, NS = 2, 16
NW = NC * NS
CH = 128
PER_W = N // NW
IDX_ROWS = PER_W // CH
IDX_BLK = 8
N_OUTER = IDX_ROWS // IDX_BLK


def _sinusoidal_pe(max_len, d):
    pos = np.arange(max_len, dtype=np.float32)[:, None]
    div = np.exp(np.arange(0, d, 2, dtype=np.float32) * (-np.log(10000.0) / d))
    pe = np.zeros((max_len, d), dtype=np.float32)
    pe[:, 0::2] = np.sin(pos * div)
    pe[:, 1::2] = np.cos(pos * div)
    return pe


R = 1600
G = N // R
_PE_TILED = jnp.asarray(np.tile(_sinusoidal_pe(MAXLEN, D)[:L], (R // L, 1)))


def _sc_gather_body(idx_hbm, table_hbm, out_hbm, idx_v, rows_v, sem):
    wid = lax.axis_index("s") * NC + lax.axis_index("c")
    row0 = wid * IDX_ROWS

    def outer(j, carry):
        pltpu.sync_copy(idx_hbm.at[pl.ds(row0 + j * IDX_BLK, IDX_BLK)], idx_v)
        for bb in range(IDX_BLK):
            pltpu.async_copy(table_hbm.at[idx_v.at[bb]], rows_v, sem).wait()
            off = (row0 + j * IDX_BLK + bb) * CH
            pltpu.sync_copy(rows_v, out_hbm.at[pl.ds(off, CH)])
        return carry

    lax.fori_loop(0, N_OUTER, outer, 0)


@functools.cache
def _sc_gather_kernel():
    return functools.partial(
        pl.kernel,
        mesh=plsc.VectorSubcoreMesh(core_axis_name="c", subcore_axis_name="s"),
        out_type=jax.ShapeDtypeStruct((N, D), jnp.float32),
        scratch_types=[
            pltpu.VMEM((IDX_BLK, CH), jnp.int32),
            pltpu.VMEM((CH, D), jnp.float32),
            pltpu.SemaphoreType.DMA,
        ],
        compiler_params=pltpu.CompilerParams(use_tc_tiling_on_sc=False),
    )(_sc_gather_body)


_MAXN = 1.0 - 1e-5
_AMAX = float(np.arctanh(1.0 - 1e-5))


def _tc_body(emb_ref, pe_ref, w_ref, b_ref, g_ref, be_ref, out_ref):
    e = emb_ref[...] + pe_ref[...]
    ew = jnp.dot(e, w_ref[...].T, preferred_element_type=jnp.float32)
    n = jnp.sqrt(jnp.clip(jnp.sum(e * e, axis=-1, keepdims=True), 1e-15))
    nw = jnp.sqrt(jnp.clip(jnp.sum(ew * ew, axis=-1, keepdims=True), 1e-15))
    s2 = jnp.minimum(jnp.tanh(nw / n * jnp.minimum(n, _AMAX)), _MAXN)
    bv = b_ref[...]
    bn = jnp.sqrt(jnp.clip(jnp.sum(bv * bv, axis=-1, keepdims=True), 1e-15))
    bh = jnp.minimum(jnp.tanh(bn), _MAXN) * bv / bn
    y2 = jnp.sum(bh * bh, axis=-1, keepdims=True)
    xy = s2 * jnp.sum(ew * bh, axis=-1, keepdims=True) / nw
    x2 = s2 * s2
    h = ((1.0 + 2.0 * xy + y2) * (s2 / nw)) * ew + (1.0 - x2) * bh
    mu = jnp.mean(h, axis=-1, keepdims=True)
    hc = h - mu
    var = jnp.mean(hc * hc, axis=-1, keepdims=True)
    out_ref[...] = hc / jnp.sqrt(var + EPS) * g_ref[...] + be_ref[...]


def _tc_compute(emb, pe_t, W, b2, g2, be2):
    return pl.pallas_call(
        _tc_body,
        grid=(G,),
        in_specs=[
            pl.BlockSpec((R, D), lambda i: (i, 0)),
            pl.BlockSpec((R, D), lambda i: (0, 0)),
            pl.BlockSpec((D, D), lambda i: (0, 0)),
            pl.BlockSpec((1, D), lambda i: (0, 0)),
            pl.BlockSpec((1, D), lambda i: (0, 0)),
            pl.BlockSpec((1, D), lambda i: (0, 0)),
        ],
        out_specs=pl.BlockSpec((R, D), lambda i: (i, 0)),
        out_shape=jax.ShapeDtypeStruct((N, D), jnp.float32),
        compiler_params=pltpu.CompilerParams(
            dimension_semantics=("arbitrary",),
        ),
    )(emb, pe_t, W, b2, g2, be2)


def kernel(x, table, W, b, gamma, beta):
    xf = x.reshape(N // CH, CH).astype(jnp.int32)
    emb = _sc_gather_kernel()(xf, table)
    out = _tc_compute(
        emb,
        _PE_TILED,
        W,
        b.reshape(1, D),
        gamma.reshape(1, D),
        beta.reshape(1, D),
    )
    return out.reshape(B, L, D)
